# Initial kernel scaffold; baseline (speedup 1.0000x reference)
import jax
import jax.numpy as jnp
from jax.experimental import pallas as pl


def kernel(x, Ws1, Wn1, b1, Ws2, Wn2, b2, Ws3, Wn3, b3, PW1, Pb1, PW2, Pb2, edge_index, neg_edge_index):
    raise NotImplementedError("write your pallas kernel here")


# speedup vs baseline: 1.6157x; 1.6157x over previous
import functools

import jax
import jax.numpy as jnp
from jax import lax
from jax.experimental import pallas as pl
from jax.experimental.pallas import tpu as pltpu
from jax.experimental.pallas import tpu_sc as plsc

N = 10000
E = 320000
D = 128
NC = 2
NS = 16
NW = NC * NS
EPW = E // NW
K = 80
NCH = EPW // K
HN = N // NC
ACC_ROWS = 5008
EPC = E // NS
NCHC = EPC // K
RZ = 312
ZTAIL = ACC_ROWS - NS * RZ
OTAIL = HN - NS * RZ
SUB = 104

@functools.lru_cache(maxsize=None)
def _mesh():
    return plsc.VectorSubcoreMesh(core_axis_name="c", subcore_axis_name="s")


def _agg_body(with_deg, h, srcv, dstv, z128, ones_h, *refs):
    if with_deg:
        (outH, outD, sidx, didx, didx2, rows, onesv, bounce,
         accH, accD, sem) = refs
    else:
        (outH, sidx, didx, didx2, rows, bounce, accH, sem) = refs
    c = lax.axis_index("c")
    s = lax.axis_index("s")
    half0 = c * HN
    r0 = s * RZ
    pltpu.sync_copy(z128, bounce)

    def zstep(j, carry):
        pltpu.sync_copy(bounce, accH.at[pl.ds(r0 + j * SUB, SUB)])
        if with_deg:
            pltpu.sync_copy(bounce, accD.at[pl.ds(r0 + j * SUB, SUB)])
        return carry

    lax.fori_loop(0, RZ // SUB, zstep, 0)
    if with_deg:
        pltpu.sync_copy(ones_h, onesv)

    @pl.when(s == NS - 1)
    def _():
        pltpu.sync_copy(bounce.at[pl.ds(0, ZTAIL)], accH.at[pl.ds(NS * RZ, ZTAIL)])
        if with_deg:
            pltpu.sync_copy(bounce.at[pl.ds(0, ZTAIL)], accD.at[pl.ds(NS * RZ, ZTAIL)])

    plsc.subcore_barrier()

    base0 = s * EPC

    def step(i, carry):
        b = base0 + i * K
        pltpu.sync_copy(srcv.at[pl.ds(b, K)], sidx)
        pltpu.sync_copy(dstv.at[pl.ds(b, K)], didx)
        for g in range(K // 16):
            v = didx[pl.ds(g * 16, 16)]
            lv = v - half0
            ok = (lv >= 0) & (lv < HN)
            didx2[pl.ds(g * 16, 16)] = jnp.where(ok, lv, HN)
        pltpu.async_copy(h.at[sidx], rows, sem).wait()
        pltpu.sync_copy(rows, accH.at[didx2], add=True)
        if with_deg:
            pltpu.sync_copy(onesv, accD.at[didx2], add=True)
        return carry

    lax.fori_loop(0, NCHC, step, 0)
    plsc.subcore_barrier()

    def ostep(j, carry):
        rj = r0 + j * SUB
        pltpu.sync_copy(accH.at[pl.ds(rj, SUB)], bounce)
        pltpu.sync_copy(bounce, outH.at[pl.ds(half0 + rj, SUB)])
        if with_deg:
            pltpu.sync_copy(accD.at[pl.ds(rj, SUB)], bounce)
            pltpu.sync_copy(bounce, outD.at[pl.ds(half0 + rj, SUB)])
        return carry

    lax.fori_loop(0, RZ // SUB, ostep, 0)

    @pl.when(s == NS - 1)
    def _():
        pltpu.sync_copy(accH.at[pl.ds(NS * RZ, OTAIL)], bounce.at[pl.ds(0, OTAIL)])
        pltpu.sync_copy(bounce.at[pl.ds(0, OTAIL)], outH.at[pl.ds(half0 + NS * RZ, OTAIL)])
        if with_deg:
            pltpu.sync_copy(accD.at[pl.ds(NS * RZ, OTAIL)], bounce.at[pl.ds(0, OTAIL)])
            pltpu.sync_copy(bounce.at[pl.ds(0, OTAIL)], outD.at[pl.ds(half0 + NS * RZ, OTAIL)])


@functools.lru_cache(maxsize=None)
def _make_agg(with_deg):
    out_type = [jax.ShapeDtypeStruct((N, D), jnp.float32)]
    scratch = [
        pltpu.VMEM((K,), jnp.int32),
        pltpu.VMEM((K,), jnp.int32),
        pltpu.VMEM((K,), jnp.int32),
        pltpu.VMEM((K, D), jnp.float32),
    ]
    if with_deg:
        out_type.append(jax.ShapeDtypeStruct((N, D), jnp.float32))
        scratch.insert(4, pltpu.VMEM((K, D), jnp.float32))
    scratch.append(pltpu.VMEM((SUB, D), jnp.float32))
    scratch.append(pltpu.VMEM_SHARED((ACC_ROWS, D), jnp.float32))
    if with_deg:
        scratch.append(pltpu.VMEM_SHARED((ACC_ROWS, D), jnp.float32))
    scratch.append(pltpu.SemaphoreType.DMA)
    return pl.kernel(
        functools.partial(_agg_body, with_deg),
        mesh=_mesh(),
        out_type=out_type,
        scratch_types=scratch,
        compiler_params=pltpu.CompilerParams(needs_layout_passes=False),
    )


def _pred_body(av, bv, w2v_h, psrc, pdst, nsrc, ndst, pout, nout,
               sidx, didx, rows_a, rows_b, outv, w2v, sem):
    c = lax.axis_index("c")
    s = lax.axis_index("s")
    wid = c * NS + s
    pltpu.sync_copy(w2v_h, w2v)
    pb2 = w2v[pl.ds(D, 16)][0]
    base0 = wid * EPW
    iota = lax.iota(jnp.int32, 16)

    for (srcr, dstr, outr) in ((psrc, pdst, pout), (nsrc, ndst, nout)):
        def step(i, carry):
            b = base0 + i * K
            pltpu.sync_copy(srcr.at[pl.ds(b, K)], sidx)
            pltpu.sync_copy(dstr.at[pl.ds(b, K)], didx)
            pltpu.async_copy(av.at[sidx], rows_a, sem).wait()
            pltpu.async_copy(bv.at[didx], rows_b, sem).wait()
            for g in range(K // 16):
                rid = iota + g * 16

                def kvstep(kv, acc):
                    wv = w2v[pl.ds(kv * 16, 16)]
                    k0 = kv * 16
                    for j in range(16):
                        cid = jnp.zeros((16,), jnp.int32) + (k0 + j)
                        va = plsc.load_gather(rows_a, [rid, cid])
                        vb = plsc.load_gather(rows_b, [rid, cid])
                        acc = acc + jnp.maximum(va + vb, 0.0) * wv[j]
                    return acc

                acc0 = jnp.zeros((16,), jnp.float32) + pb2
                acc = lax.fori_loop(0, D // 16, kvstep, acc0)
                outv[pl.ds(g * 16, 16)] = acc
            pltpu.sync_copy(outv, outr.at[pl.ds(b, K)])
            return carry

        lax.fori_loop(0, NCH, step, 0)


@functools.lru_cache(maxsize=None)
def _make_pred():
    return pl.kernel(
        _pred_body,
        mesh=_mesh(),
        out_type=[
            jax.ShapeDtypeStruct((E,), jnp.float32),
            jax.ShapeDtypeStruct((E,), jnp.float32),
        ],
        scratch_types=[
            pltpu.VMEM((K,), jnp.int32),
            pltpu.VMEM((K,), jnp.int32),
            pltpu.VMEM((K, D), jnp.float32),
            pltpu.VMEM((K, D), jnp.float32),
            pltpu.VMEM((K,), jnp.float32),
            pltpu.VMEM((D + 16,), jnp.float32),
            pltpu.SemaphoreType.DMA,
        ],
        compiler_params=pltpu.CompilerParams(needs_layout_passes=False),
    )


_RB = 1000


def _mean(agg_ref, deg_ref):
    dsum = deg_ref[:, 0:1]
    return agg_ref[...] * (1.0 / jnp.maximum(dsum, 1.0))


def _layer_body(h_ref, agg_ref, deg_ref, ws_ref, wn_ref, b_ref, o_ref):
    mean = _mean(agg_ref, deg_ref)
    o_ref[...] = (
        jnp.dot(h_ref[...], ws_ref[...], preferred_element_type=jnp.float32)
        + jnp.dot(mean, wn_ref[...], preferred_element_type=jnp.float32)
        + b_ref[...]
    )


def _layer3_body(h_ref, agg_ref, deg_ref, ws_ref, wn_ref, b_ref,
                 pa_ref, pb_ref, pb1_ref, a_ref, bm_ref):
    mean = _mean(agg_ref, deg_ref)
    h3 = (
        jnp.dot(h_ref[...], ws_ref[...], preferred_element_type=jnp.float32)
        + jnp.dot(mean, wn_ref[...], preferred_element_type=jnp.float32)
        + b_ref[...]
    )
    a_ref[...] = jnp.dot(h3, pa_ref[...], preferred_element_type=jnp.float32) + pb1_ref[...]
    bm_ref[...] = jnp.dot(h3, pb_ref[...], preferred_element_type=jnp.float32)


_w_spec = pl.BlockSpec((D, D), lambda i: (0, 0))
_b_spec = pl.BlockSpec((1, D), lambda i: (0, 0))
_h_spec = pl.BlockSpec((_RB, D), lambda i: (i, 0))
_agg_spec = pl.BlockSpec((_RB, D), lambda i: (i, 0))
_deg_spec = pl.BlockSpec((_RB, D), lambda i: (i, 0))

_layer_call = pl.pallas_call(
    _layer_body,
    grid=(N // _RB,),
    in_specs=[_h_spec, _agg_spec, _deg_spec, _w_spec, _w_spec, _b_spec],
    out_specs=_h_spec,
    out_shape=jax.ShapeDtypeStruct((N, D), jnp.float32),
)

_layer3_call = pl.pallas_call(
    _layer3_body,
    grid=(N // _RB,),
    in_specs=[_h_spec, _agg_spec, _deg_spec, _w_spec, _w_spec, _b_spec,
              _w_spec, _w_spec, _b_spec],
    out_specs=[_h_spec, _h_spec],
    out_shape=[jax.ShapeDtypeStruct((N, D), jnp.float32),
               jax.ShapeDtypeStruct((N, D), jnp.float32)],
)


@jax.jit
def kernel(x, Ws1, Wn1, b1, Ws2, Wn2, b2, Ws3, Wn3, b3,
           PW1, Pb1, PW2, Pb2, edge_index, neg_edge_index):
    src = edge_index[0]
    dst = edge_index[1]
    z128 = jnp.zeros((SUB, D), jnp.float32)
    ones_h = jnp.ones((K, D), jnp.float32)

    agg1, degp = _make_agg(True)(x, src, dst, z128, ones_h)
    h1 = _layer_call(x, agg1, degp, Ws1, Wn1, b1.reshape(1, D))
    (agg2,) = _make_agg(False)(h1, src, dst, z128, ones_h)
    h2 = _layer_call(h1, agg2, degp, Ws2, Wn2, b2.reshape(1, D))
    (agg3,) = _make_agg(False)(h2, src, dst, z128, ones_h)
    a, bm = _layer3_call(h2, agg3, degp, Ws3, Wn3, b3.reshape(1, D),
                         PW1[:D], PW1[D:], Pb1.reshape(1, D))

    w2pad = jnp.concatenate([PW2[:, 0], Pb2, jnp.zeros((15,), jnp.float32)])
    pos, neg = _make_pred()(a, bm, w2pad,
                          edge_index[0], edge_index[1],
                          neg_edge_index[0], neg_edge_index[1])
    return pos, neg


# speedup vs baseline: 2.3523x; 1.4559x over previous
import functools

import jax
import jax.numpy as jnp
from jax import lax
from jax.experimental import pallas as pl
from jax.experimental.pallas import tpu as pltpu
from jax.experimental.pallas import tpu_sc as plsc

N = 10000
E = 320000
D = 128
NC = 2
NS = 16
NW = NC * NS
EPW = E // NW
K = 80
NCH = EPW // K
HN = N // NC
ACC_ROWS = 5008
EPC = E // NS
NCHC = EPC // K
RZ = 312
ZTAIL = ACC_ROWS - NS * RZ
OTAIL = HN - NS * RZ
SUB = 24
SB = 2000
CPB = SB // K
PAIRS = (CPB - 1) // 2

@functools.lru_cache(maxsize=None)
def _mesh():
    return plsc.VectorSubcoreMesh(core_axis_name="c", subcore_axis_name="s")


def _agg_body(with_deg, h, srcv, dstv, z128, ones_h, *refs):
    if with_deg:
        (outH, outD, sblk, dblk, didx2, rows0, rows1, onesv, bounce,
         accH, accD, sem0, sem1) = refs
    else:
        (outH, sblk, dblk, didx2, rows0, rows1, bounce, accH, sem0, sem1) = refs
    c = lax.axis_index("c")
    s = lax.axis_index("s")
    half0 = c * HN
    r0 = s * RZ
    pltpu.sync_copy(z128, bounce)

    def zstep(j, carry):
        pltpu.sync_copy(bounce, accH.at[pl.ds(r0 + j * SUB, SUB)])
        if with_deg:
            pltpu.sync_copy(bounce, accD.at[pl.ds(r0 + j * SUB, SUB)])
        return carry

    lax.fori_loop(0, RZ // SUB, zstep, 0)
    if with_deg:
        pltpu.sync_copy(ones_h, onesv)

    @pl.when(s == NS - 1)
    def _():
        pltpu.sync_copy(bounce.at[pl.ds(0, ZTAIL)], accH.at[pl.ds(NS * RZ, ZTAIL)])
        if with_deg:
            pltpu.sync_copy(bounce.at[pl.ds(0, ZTAIL)], accD.at[pl.ds(NS * RZ, ZTAIL)])

    plsc.subcore_barrier()

    base0 = s * EPC

    def route(ofs):
        for g in range(K // 16):
            v = dblk[pl.ds(ofs + g * 16, 16)]
            lv = v - half0
            ok = (lv >= 0) & (lv < HN)
            didx2[pl.ds(g * 16, 16)] = jnp.where(ok, lv, HN)

    def scat(ofs, rowbuf):
        route(ofs)
        pltpu.sync_copy(rowbuf, accH.at[didx2], add=True)
        if with_deg:
            pltpu.sync_copy(onesv, accD.at[didx2], add=True)

    def wait_on(buf, sem):
        pltpu.make_async_copy(h.at[pl.ds(0, K)], buf, sem).wait()

    def sbstep(sb, carry):
        blk = base0 + sb * SB
        pltpu.sync_copy(srcv.at[pl.ds(blk, SB)], sblk)
        pltpu.sync_copy(dstv.at[pl.ds(blk, SB)], dblk)
        pltpu.async_copy(h.at[sblk.at[pl.ds(0, K)]], rows0, sem0)

        def pair(m, carry2):
            o = 2 * m * K
            pltpu.async_copy(h.at[sblk.at[pl.ds(o + K, K)]], rows1, sem1)
            wait_on(rows0, sem0)
            scat(o, rows0)
            pltpu.async_copy(h.at[sblk.at[pl.ds(o + 2 * K, K)]], rows0, sem0)
            wait_on(rows1, sem1)
            scat(o + K, rows1)
            return carry2

        lax.fori_loop(0, PAIRS, pair, 0)
        wait_on(rows0, sem0)
        scat((CPB - 1) * K, rows0)
        return carry

    lax.fori_loop(0, EPC // SB, sbstep, 0)
    plsc.subcore_barrier()

    def ostep(j, carry):
        rj = r0 + j * SUB
        pltpu.sync_copy(accH.at[pl.ds(rj, SUB)], bounce)
        pltpu.sync_copy(bounce, outH.at[pl.ds(half0 + rj, SUB)])
        if with_deg:
            pltpu.sync_copy(accD.at[pl.ds(rj, SUB)], bounce)
            pltpu.sync_copy(bounce, outD.at[pl.ds(half0 + rj, SUB)])
        return carry

    lax.fori_loop(0, RZ // SUB, ostep, 0)

    @pl.when(s == NS - 1)
    def _():
        pltpu.sync_copy(accH.at[pl.ds(NS * RZ, OTAIL)], bounce.at[pl.ds(0, OTAIL)])
        pltpu.sync_copy(bounce.at[pl.ds(0, OTAIL)], outH.at[pl.ds(half0 + NS * RZ, OTAIL)])
        if with_deg:
            pltpu.sync_copy(accD.at[pl.ds(NS * RZ, OTAIL)], bounce.at[pl.ds(0, OTAIL)])
            pltpu.sync_copy(bounce.at[pl.ds(0, OTAIL)], outD.at[pl.ds(half0 + NS * RZ, OTAIL)])


@functools.lru_cache(maxsize=None)
def _make_agg(with_deg):
    out_type = [jax.ShapeDtypeStruct((N, D), jnp.float32)]
    scratch = [
        pltpu.VMEM((SB,), jnp.int32),
        pltpu.VMEM((SB,), jnp.int32),
        pltpu.VMEM((K,), jnp.int32),
        pltpu.VMEM((K, D), jnp.float32),
        pltpu.VMEM((K, D), jnp.float32),
    ]
    if with_deg:
        out_type.append(jax.ShapeDtypeStruct((N, D), jnp.float32))
        scratch.insert(5, pltpu.VMEM((K, D), jnp.float32))
    scratch.append(pltpu.VMEM((SUB, D), jnp.float32))
    scratch.append(pltpu.VMEM_SHARED((ACC_ROWS, D), jnp.float32))
    if with_deg:
        scratch.append(pltpu.VMEM_SHARED((ACC_ROWS, D), jnp.float32))
    scratch.append(pltpu.SemaphoreType.DMA)
    scratch.append(pltpu.SemaphoreType.DMA)
    return pl.kernel(
        functools.partial(_agg_body, with_deg),
        mesh=_mesh(),
        out_type=out_type,
        scratch_types=scratch,
        compiler_params=pltpu.CompilerParams(needs_layout_passes=False),
    )


def _pred_body(av, bv, w2v_h, psrc, pdst, nsrc, ndst, pout, nout,
               sblk, dblk, ra0, ra1, rb0, rb1, outblk, w2v, sem0, sem1):
    c = lax.axis_index("c")
    s = lax.axis_index("s")
    wid = c * NS + s
    pltpu.sync_copy(w2v_h, w2v)
    pb2 = w2v[pl.ds(D, 16)][0]
    base0 = wid * EPW
    iota = lax.iota(jnp.int32, 16)

    def compute(o, ra, rb):
        for g in range(K // 16):
            rid = iota + g * 16

            def kvstep(kv, acc):
                wv = w2v[pl.ds(kv * 16, 16)]
                k0 = kv * 16
                for j in range(16):
                    cid = jnp.zeros((16,), jnp.int32) + (k0 + j)
                    va = plsc.load_gather(ra, [rid, cid])
                    vb = plsc.load_gather(rb, [rid, cid])
                    acc = acc + jnp.maximum(va + vb, 0.0) * wv[j]
                return acc

            acc0 = jnp.zeros((16,), jnp.float32) + pb2
            acc = lax.fori_loop(0, D // 16, kvstep, acc0)
            outblk[pl.ds(o + g * 16, 16)] = acc

    def start(o, ra, rb, sem):
        pltpu.async_copy(av.at[sblk.at[pl.ds(o, K)]], ra, sem)
        pltpu.async_copy(bv.at[dblk.at[pl.ds(o, K)]], rb, sem)

    def wait_pair(ra, rb, sem):
        pltpu.make_async_copy(av.at[pl.ds(0, K)], ra, sem).wait()
        pltpu.make_async_copy(av.at[pl.ds(0, K)], rb, sem).wait()

    for (srcr, dstr, outr) in ((psrc, pdst, pout), (nsrc, ndst, nout)):
        def sbstep(sb, carry):
            blk = base0 + sb * SB
            pltpu.sync_copy(srcr.at[pl.ds(blk, SB)], sblk)
            pltpu.sync_copy(dstr.at[pl.ds(blk, SB)], dblk)
            start(0, ra0, rb0, sem0)

            def pair(m, carry2):
                o = 2 * m * K
                start(o + K, ra1, rb1, sem1)
                wait_pair(ra0, rb0, sem0)
                compute(o, ra0, rb0)
                start(o + 2 * K, ra0, rb0, sem0)
                wait_pair(ra1, rb1, sem1)
                compute(o + K, ra1, rb1)
                return carry2

            lax.fori_loop(0, PAIRS, pair, 0)
            wait_pair(ra0, rb0, sem0)
            compute((CPB - 1) * K, ra0, rb0)
            pltpu.sync_copy(outblk, outr.at[pl.ds(blk, SB)])
            return carry

        lax.fori_loop(0, EPW // SB, sbstep, 0)


@functools.lru_cache(maxsize=None)
def _make_pred():
    return pl.kernel(
        _pred_body,
        mesh=_mesh(),
        out_type=[
            jax.ShapeDtypeStruct((E,), jnp.float32),
            jax.ShapeDtypeStruct((E,), jnp.float32),
        ],
        scratch_types=[
            pltpu.VMEM((SB,), jnp.int32),
            pltpu.VMEM((SB,), jnp.int32),
            pltpu.VMEM((K, D), jnp.float32),
            pltpu.VMEM((K, D), jnp.float32),
            pltpu.VMEM((K, D), jnp.float32),
            pltpu.VMEM((K, D), jnp.float32),
            pltpu.VMEM((SB,), jnp.float32),
            pltpu.VMEM((D + 16,), jnp.float32),
            pltpu.SemaphoreType.DMA,
            pltpu.SemaphoreType.DMA,
        ],
        compiler_params=pltpu.CompilerParams(needs_layout_passes=False),
    )


_RB = 1000


def _mean(agg_ref, deg_ref):
    dsum = deg_ref[:, 0:1]
    return agg_ref[...] * (1.0 / jnp.maximum(dsum, 1.0))


def _layer_body(h_ref, agg_ref, deg_ref, ws_ref, wn_ref, b_ref, o_ref):
    mean = _mean(agg_ref, deg_ref)
    o_ref[...] = (
        jnp.dot(h_ref[...], ws_ref[...], preferred_element_type=jnp.float32)
        + jnp.dot(mean, wn_ref[...], preferred_element_type=jnp.float32)
        + b_ref[...]
    )


def _layer3_body(h_ref, agg_ref, deg_ref, ws_ref, wn_ref, b_ref,
                 pa_ref, pb_ref, pb1_ref, a_ref, bm_ref):
    mean = _mean(agg_ref, deg_ref)
    h3 = (
        jnp.dot(h_ref[...], ws_ref[...], preferred_element_type=jnp.float32)
        + jnp.dot(mean, wn_ref[...], preferred_element_type=jnp.float32)
        + b_ref[...]
    )
    a_ref[...] = jnp.dot(h3, pa_ref[...], preferred_element_type=jnp.float32) + pb1_ref[...]
    bm_ref[...] = jnp.dot(h3, pb_ref[...], preferred_element_type=jnp.float32)


_w_spec = pl.BlockSpec((D, D), lambda i: (0, 0))
_b_spec = pl.BlockSpec((1, D), lambda i: (0, 0))
_h_spec = pl.BlockSpec((_RB, D), lambda i: (i, 0))
_agg_spec = pl.BlockSpec((_RB, D), lambda i: (i, 0))
_deg_spec = pl.BlockSpec((_RB, D), lambda i: (i, 0))

_layer_call = pl.pallas_call(
    _layer_body,
    grid=(N // _RB,),
    in_specs=[_h_spec, _agg_spec, _deg_spec, _w_spec, _w_spec, _b_spec],
    out_specs=_h_spec,
    out_shape=jax.ShapeDtypeStruct((N, D), jnp.float32),
)

_layer3_call = pl.pallas_call(
    _layer3_body,
    grid=(N // _RB,),
    in_specs=[_h_spec, _agg_spec, _deg_spec, _w_spec, _w_spec, _b_spec,
              _w_spec, _w_spec, _b_spec],
    out_specs=[_h_spec, _h_spec],
    out_shape=[jax.ShapeDtypeStruct((N, D), jnp.float32),
               jax.ShapeDtypeStruct((N, D), jnp.float32)],
)


@jax.jit
def kernel(x, Ws1, Wn1, b1, Ws2, Wn2, b2, Ws3, Wn3, b3,
           PW1, Pb1, PW2, Pb2, edge_index, neg_edge_index):
    src = edge_index[0]
    dst = edge_index[1]
    z128 = jnp.zeros((SUB, D), jnp.float32)
    ones_h = jnp.ones((K, D), jnp.float32)

    agg1, degp = _make_agg(True)(x, src, dst, z128, ones_h)
    h1 = _layer_call(x, agg1, degp, Ws1, Wn1, b1.reshape(1, D))
    (agg2,) = _make_agg(False)(h1, src, dst, z128, ones_h)
    h2 = _layer_call(h1, agg2, degp, Ws2, Wn2, b2.reshape(1, D))
    (agg3,) = _make_agg(False)(h2, src, dst, z128, ones_h)
    a, bm = _layer3_call(h2, agg3, degp, Ws3, Wn3, b3.reshape(1, D),
                         PW1[:D], PW1[D:], Pb1.reshape(1, D))

    w2pad = jnp.concatenate([PW2[:, 0], Pb2, jnp.zeros((15,), jnp.float32)])
    pos, neg = _make_pred()(a, bm, w2pad,
                          edge_index[0], edge_index[1],
                          neg_edge_index[0], neg_edge_index[1])
    return pos, neg


# speedup vs baseline: 2.5143x; 1.0689x over previous
import functools

import jax
import jax.numpy as jnp
from jax import lax
from jax.experimental import pallas as pl
from jax.experimental.pallas import tpu as pltpu
from jax.experimental.pallas import tpu_sc as plsc

N = 10000
E = 320000
D = 128
NC = 2
NS = 16
NW = NC * NS
EPW = E // NW
K = 80
NCH = EPW // K
HN = N // NC
ACC_ROWS = 5008
EPC = E // NS
NCHC = EPC // K
RZ = 312
ZTAIL = ACC_ROWS - NS * RZ
OTAIL = HN - NS * RZ
SUB = 24
SB = 2000
CPB = SB // K
PAIRS = (CPB - 1) // 2

@functools.lru_cache(maxsize=None)
def _mesh():
    return plsc.VectorSubcoreMesh(core_axis_name="c", subcore_axis_name="s")


def _agg_body(with_deg, h, srcv, dstv, z128, ones_h, *refs):
    if with_deg:
        (outH, outD, sblk, dblk, didx2, rows0, rows1, onesv, bounce,
         accH, accD, sem0, sem1) = refs
    else:
        (outH, sblk, dblk, didx2, rows0, rows1, bounce, accH, sem0, sem1) = refs
    c = lax.axis_index("c")
    s = lax.axis_index("s")
    half0 = c * HN
    r0 = s * RZ
    pltpu.sync_copy(z128, bounce)

    def zstep(j, carry):
        pltpu.sync_copy(bounce, accH.at[pl.ds(r0 + j * SUB, SUB)])
        if with_deg:
            pltpu.sync_copy(bounce, accD.at[pl.ds(r0 + j * SUB, SUB)])
        return carry

    lax.fori_loop(0, RZ // SUB, zstep, 0)
    if with_deg:
        pltpu.sync_copy(ones_h, onesv)

    @pl.when(s == NS - 1)
    def _():
        pltpu.sync_copy(bounce.at[pl.ds(0, ZTAIL)], accH.at[pl.ds(NS * RZ, ZTAIL)])
        if with_deg:
            pltpu.sync_copy(bounce.at[pl.ds(0, ZTAIL)], accD.at[pl.ds(NS * RZ, ZTAIL)])

    plsc.subcore_barrier()

    base0 = s * EPC

    def route(ofs):
        for g in range(K // 16):
            v = dblk[pl.ds(ofs + g * 16, 16)]
            lv = v - half0
            ok = (lv >= 0) & (lv < HN)
            didx2[pl.ds(g * 16, 16)] = jnp.where(ok, lv, HN)

    def scat(ofs, rowbuf):
        route(ofs)
        pltpu.sync_copy(rowbuf, accH.at[didx2], add=True)
        if with_deg:
            pltpu.sync_copy(onesv, accD.at[didx2], add=True)

    def wait_on(buf, sem):
        pltpu.make_async_copy(h.at[pl.ds(0, K)], buf, sem).wait()

    def sbstep(sb, carry):
        blk = base0 + sb * SB
        pltpu.sync_copy(srcv.at[pl.ds(blk, SB)], sblk)
        pltpu.sync_copy(dstv.at[pl.ds(blk, SB)], dblk)
        pltpu.async_copy(h.at[sblk.at[pl.ds(0, K)]], rows0, sem0)

        def pair(m, carry2):
            o = 2 * m * K
            pltpu.async_copy(h.at[sblk.at[pl.ds(o + K, K)]], rows1, sem1)
            wait_on(rows0, sem0)
            scat(o, rows0)
            pltpu.async_copy(h.at[sblk.at[pl.ds(o + 2 * K, K)]], rows0, sem0)
            wait_on(rows1, sem1)
            scat(o + K, rows1)
            return carry2

        lax.fori_loop(0, PAIRS, pair, 0)
        wait_on(rows0, sem0)
        scat((CPB - 1) * K, rows0)
        return carry

    lax.fori_loop(0, EPC // SB, sbstep, 0)
    plsc.subcore_barrier()

    def ostep(j, carry):
        rj = r0 + j * SUB
        pltpu.sync_copy(accH.at[pl.ds(rj, SUB)], bounce)
        pltpu.sync_copy(bounce, outH.at[pl.ds(half0 + rj, SUB)])
        if with_deg:
            pltpu.sync_copy(accD.at[pl.ds(rj, SUB)], bounce)
            pltpu.sync_copy(bounce, outD.at[pl.ds(half0 + rj, SUB)])
        return carry

    lax.fori_loop(0, RZ // SUB, ostep, 0)

    @pl.when(s == NS - 1)
    def _():
        pltpu.sync_copy(accH.at[pl.ds(NS * RZ, OTAIL)], bounce.at[pl.ds(0, OTAIL)])
        pltpu.sync_copy(bounce.at[pl.ds(0, OTAIL)], outH.at[pl.ds(half0 + NS * RZ, OTAIL)])
        if with_deg:
            pltpu.sync_copy(accD.at[pl.ds(NS * RZ, OTAIL)], bounce.at[pl.ds(0, OTAIL)])
            pltpu.sync_copy(bounce.at[pl.ds(0, OTAIL)], outD.at[pl.ds(half0 + NS * RZ, OTAIL)])


@functools.lru_cache(maxsize=None)
def _make_agg(with_deg):
    out_type = [jax.ShapeDtypeStruct((N, D), jnp.float32)]
    scratch = [
        pltpu.VMEM((SB,), jnp.int32),
        pltpu.VMEM((SB,), jnp.int32),
        pltpu.VMEM((K,), jnp.int32),
        pltpu.VMEM((K, D), jnp.float32),
        pltpu.VMEM((K, D), jnp.float32),
    ]
    if with_deg:
        out_type.append(jax.ShapeDtypeStruct((N, D), jnp.float32))
        scratch.insert(5, pltpu.VMEM((K, D), jnp.float32))
    scratch.append(pltpu.VMEM((SUB, D), jnp.float32))
    scratch.append(pltpu.VMEM_SHARED((ACC_ROWS, D), jnp.float32))
    if with_deg:
        scratch.append(pltpu.VMEM_SHARED((ACC_ROWS, D), jnp.float32))
    scratch.append(pltpu.SemaphoreType.DMA)
    scratch.append(pltpu.SemaphoreType.DMA)
    return pl.kernel(
        functools.partial(_agg_body, with_deg),
        mesh=_mesh(),
        out_type=out_type,
        scratch_types=scratch,
        compiler_params=pltpu.CompilerParams(needs_layout_passes=False),
    )


def _pred_body(av, bv, w2v_h, psrc, pdst, nsrc, ndst, pout, nout,
               sblk, dblk, ra0, ra1, rb0, rb1, outblk, w2v, sem0, sem1):
    c = lax.axis_index("c")
    s = lax.axis_index("s")
    wid = c * NS + s
    pltpu.sync_copy(w2v_h, w2v)
    pb2 = w2v[pl.ds(D, 16)][0]
    base0 = wid * EPW
    iota = lax.iota(jnp.int32, 16)

    def compute(o, ra, rb):
        for g in range(K // 16):
            rid = iota + g * 16

            def kvstep(kv, accs):
                wv = w2v[pl.ds(kv * 16, 16)]
                k0 = kv * 16
                accs = list(accs)
                for j in range(16):
                    cid = jnp.zeros((16,), jnp.int32) + (k0 + j)
                    va = plsc.load_gather(ra, [rid, cid])
                    vb = plsc.load_gather(rb, [rid, cid])
                    p = j % 4
                    accs[p] = accs[p] + jnp.maximum(va + vb, 0.0) * wv[j]
                return tuple(accs)

            z = jnp.zeros((16,), jnp.float32)
            a0, a1, a2, a3 = lax.fori_loop(
                0, D // 16, kvstep, (z + pb2, z, z, z))
            outblk[pl.ds(o + g * 16, 16)] = (a0 + a1) + (a2 + a3)

    def start(o, ra, rb, sem):
        pltpu.async_copy(av.at[sblk.at[pl.ds(o, K)]], ra, sem)
        pltpu.async_copy(bv.at[dblk.at[pl.ds(o, K)]], rb, sem)

    def wait_pair(ra, rb, sem):
        pltpu.make_async_copy(av.at[pl.ds(0, K)], ra, sem).wait()
        pltpu.make_async_copy(av.at[pl.ds(0, K)], rb, sem).wait()

    for (srcr, dstr, outr) in ((psrc, pdst, pout), (nsrc, ndst, nout)):
        def sbstep(sb, carry):
            blk = base0 + sb * SB
            pltpu.sync_copy(srcr.at[pl.ds(blk, SB)], sblk)
            pltpu.sync_copy(dstr.at[pl.ds(blk, SB)], dblk)
            start(0, ra0, rb0, sem0)

            def pair(m, carry2):
                o = 2 * m * K
                start(o + K, ra1, rb1, sem1)
                wait_pair(ra0, rb0, sem0)
                compute(o, ra0, rb0)
                start(o + 2 * K, ra0, rb0, sem0)
                wait_pair(ra1, rb1, sem1)
                compute(o + K, ra1, rb1)
                return carry2

            lax.fori_loop(0, PAIRS, pair, 0)
            wait_pair(ra0, rb0, sem0)
            compute((CPB - 1) * K, ra0, rb0)
            pltpu.sync_copy(outblk, outr.at[pl.ds(blk, SB)])
            return carry

        lax.fori_loop(0, EPW // SB, sbstep, 0)


@functools.lru_cache(maxsize=None)
def _make_pred():
    return pl.kernel(
        _pred_body,
        mesh=_mesh(),
        out_type=[
            jax.ShapeDtypeStruct((E,), jnp.float32),
            jax.ShapeDtypeStruct((E,), jnp.float32),
        ],
        scratch_types=[
            pltpu.VMEM((SB,), jnp.int32),
            pltpu.VMEM((SB,), jnp.int32),
            pltpu.VMEM((K, D), jnp.float32),
            pltpu.VMEM((K, D), jnp.float32),
            pltpu.VMEM((K, D), jnp.float32),
            pltpu.VMEM((K, D), jnp.float32),
            pltpu.VMEM((SB,), jnp.float32),
            pltpu.VMEM((D + 16,), jnp.float32),
            pltpu.SemaphoreType.DMA,
            pltpu.SemaphoreType.DMA,
        ],
        compiler_params=pltpu.CompilerParams(needs_layout_passes=False),
    )


_RB = 1000


def _mean(agg_ref, deg_ref):
    dsum = deg_ref[:, 0:1]
    return agg_ref[...] * (1.0 / jnp.maximum(dsum, 1.0))


def _layer_body(h_ref, agg_ref, deg_ref, ws_ref, wn_ref, b_ref, o_ref):
    mean = _mean(agg_ref, deg_ref)
    o_ref[...] = (
        jnp.dot(h_ref[...], ws_ref[...], preferred_element_type=jnp.float32)
        + jnp.dot(mean, wn_ref[...], preferred_element_type=jnp.float32)
        + b_ref[...]
    )


def _layer3_body(h_ref, agg_ref, deg_ref, ws_ref, wn_ref, b_ref,
                 pa_ref, pb_ref, pb1_ref, a_ref, bm_ref):
    mean = _mean(agg_ref, deg_ref)
    h3 = (
        jnp.dot(h_ref[...], ws_ref[...], preferred_element_type=jnp.float32)
        + jnp.dot(mean, wn_ref[...], preferred_element_type=jnp.float32)
        + b_ref[...]
    )
    a_ref[...] = jnp.dot(h3, pa_ref[...], preferred_element_type=jnp.float32) + pb1_ref[...]
    bm_ref[...] = jnp.dot(h3, pb_ref[...], preferred_element_type=jnp.float32)


_w_spec = pl.BlockSpec((D, D), lambda i: (0, 0))
_b_spec = pl.BlockSpec((1, D), lambda i: (0, 0))
_h_spec = pl.BlockSpec((_RB, D), lambda i: (i, 0))
_agg_spec = pl.BlockSpec((_RB, D), lambda i: (i, 0))
_deg_spec = pl.BlockSpec((_RB, D), lambda i: (i, 0))

_layer_call = pl.pallas_call(
    _layer_body,
    grid=(N // _RB,),
    in_specs=[_h_spec, _agg_spec, _deg_spec, _w_spec, _w_spec, _b_spec],
    out_specs=_h_spec,
    out_shape=jax.ShapeDtypeStruct((N, D), jnp.float32),
)

_layer3_call = pl.pallas_call(
    _layer3_body,
    grid=(N // _RB,),
    in_specs=[_h_spec, _agg_spec, _deg_spec, _w_spec, _w_spec, _b_spec,
              _w_spec, _w_spec, _b_spec],
    out_specs=[_h_spec, _h_spec],
    out_shape=[jax.ShapeDtypeStruct((N, D), jnp.float32),
               jax.ShapeDtypeStruct((N, D), jnp.float32)],
)


@jax.jit
def kernel(x, Ws1, Wn1, b1, Ws2, Wn2, b2, Ws3, Wn3, b3,
           PW1, Pb1, PW2, Pb2, edge_index, neg_edge_index):
    src = edge_index[0]
    dst = edge_index[1]
    z128 = jnp.zeros((SUB, D), jnp.float32)
    ones_h = jnp.ones((K, D), jnp.float32)

    agg1, degp = _make_agg(True)(x, src, dst, z128, ones_h)
    h1 = _layer_call(x, agg1, degp, Ws1, Wn1, b1.reshape(1, D))
    (agg2,) = _make_agg(False)(h1, src, dst, z128, ones_h)
    h2 = _layer_call(h1, agg2, degp, Ws2, Wn2, b2.reshape(1, D))
    (agg3,) = _make_agg(False)(h2, src, dst, z128, ones_h)
    a, bm = _layer3_call(h2, agg3, degp, Ws3, Wn3, b3.reshape(1, D),
                         PW1[:D], PW1[D:], Pb1.reshape(1, D))

    w2pad = jnp.concatenate([PW2[:, 0], Pb2, jnp.zeros((15,), jnp.float32)])
    pos, neg = _make_pred()(a, bm, w2pad,
                          edge_index[0], edge_index[1],
                          neg_edge_index[0], neg_edge_index[1])
    return pos, neg
